# Initial kernel scaffold; baseline (speedup 1.0000x reference)
import jax
import jax.numpy as jnp
from jax.experimental import pallas as pl


def kernel(inputs, embeddings):
    raise NotImplementedError("write your pallas kernel here")


# speedup vs baseline: 1.0148x; 1.0148x over previous
import functools

import jax
import jax.numpy as jnp
from jax import lax
from jax.experimental import pallas as pl
from jax.experimental.pallas import tpu as pltpu
from jax.experimental.pallas import tpu_sc as plsc

_L = 16


def _build_gather(n_total, vocab, dim, num_workers, chunk):
  per_w = n_total // num_workers
  n_chunks = per_w // chunk
  groups = chunk // _L

  mesh = plsc.VectorSubcoreMesh(core_axis_name="c", subcore_axis_name="s")

  @functools.partial(
      pl.kernel,
      mesh=mesh,
      out_type=jax.ShapeDtypeStruct((n_total, dim), jnp.float32),
      compiler_params=pltpu.CompilerParams(
          needs_layout_passes=False, use_tc_tiling_on_sc=False),
      scratch_types=[
          pltpu.VMEM((chunk,), jnp.int32),
          pltpu.VMEM((chunk,), jnp.int32),
          pltpu.VMEM((chunk, dim), jnp.float32),
          pltpu.SemaphoreType.DMA,
      ],
  )
  def gather_kernel(idx_hbm, table_hbm, out_hbm, idx_v, safe_v, rows_v, sem):
    nc = 2
    wid = lax.axis_index("s") * nc + lax.axis_index("c")
    base = wid * per_w

    def chunk_body(c, carry):
      cbase = base + c * chunk
      pltpu.sync_copy(idx_hbm.at[pl.ds(cbase, chunk)], idx_v)

      cnt = jnp.zeros((_L,), jnp.int32)
      for g in range(groups):
        v = idx_v[pl.ds(g * _L, _L)]
        oov = v >= vocab
        safe_v[pl.ds(g * _L, _L)] = jnp.where(
            oov, jnp.zeros((_L,), jnp.int32), v)
        cnt = cnt + jnp.where(
            oov, jnp.ones((_L,), jnp.int32), jnp.zeros((_L,), jnp.int32))
      mxs = jnp.sum(cnt)

      pltpu.async_copy(table_hbm.at[safe_v], rows_v, sem).wait()

      @pl.when(mxs > 0)
      def _zero_oov():
        zeros_f = jnp.zeros((_L,), jnp.float32)
        for g in range(groups):
          v = idx_v[pl.ds(g * _L, _L)]
          inval = v >= vocab
          rows = lax.iota(jnp.int32, _L) + g * _L
          for col in range(dim):
            plsc.store_scatter(
                rows_v,
                [rows, jnp.full((_L,), col, jnp.int32)],
                zeros_f,
                mask=inval,
            )

      pltpu.sync_copy(rows_v, out_hbm.at[pl.ds(cbase, chunk)])
      return carry

    lax.fori_loop(0, n_chunks, chunk_body, 0)

  return gather_kernel


def kernel(inputs, embeddings):
  b, h = inputs.shape
  vocab, dim = embeddings.shape
  n_total = b * h
  idx_flat = inputs.reshape(n_total)
  fn = _build_gather(n_total, vocab, dim, num_workers=32, chunk=128)
  out = fn(idx_flat, embeddings)
  return out.reshape(b, h, dim)


# speedup vs baseline: 1.7029x; 1.6781x over previous
import functools

import jax
import jax.numpy as jnp
from jax import lax
from jax.experimental import pallas as pl
from jax.experimental.pallas import tpu as pltpu
from jax.experimental.pallas import tpu_sc as plsc

_L = 16


def _build(vocab, batch, hist, dim):
  n_workers = 32
  chunk = 512
  groups = chunk // _L
  nbt = batch // 128
  nb4 = nbt // 4
  n_su = hist * nb4
  su_per_w = n_su // n_workers
  per_w_idx = su_per_w * chunk
  ftiles = dim // 8

  mesh = plsc.VectorSubcoreMesh(core_axis_name="c", subcore_axis_name="s")

  @functools.partial(
      pl.kernel,
      mesh=mesh,
      out_type=jax.ShapeDtypeStruct((hist, ftiles, nbt, 8 * 128), jnp.float32),
      compiler_params=pltpu.CompilerParams(
          needs_layout_passes=False, use_tc_tiling_on_sc=False),
      scratch_types=[
          pltpu.VMEM((per_w_idx,), jnp.int32),
          pltpu.VMEM((4, 128), jnp.int32),
          pltpu.VMEM((4, 128), jnp.int32),
          pltpu.VMEM((chunk, dim), jnp.float32),
          pltpu.VMEM((chunk, dim), jnp.float32),
          pltpu.VMEM((ftiles, 4, 1024), jnp.float32),
          pltpu.VMEM((ftiles, 4, 1024), jnp.float32),
          pltpu.SemaphoreType.DMA,
          pltpu.SemaphoreType.DMA,
          pltpu.SemaphoreType.DMA,
          pltpu.SemaphoreType.DMA,
          pltpu.SemaphoreType.DMA,
      ],
  )
  def gather_kernel(idx_hbm, table_hbm, out_hbm, idx_all, safe_a, safe_b,
                    rows_a, rows_b, tbuf_a, tbuf_b, isem, gsem_a, gsem_b,
                    osem_a, osem_b):
    nc = 2
    wid = lax.axis_index("s") * nc + lax.axis_index("c")
    iota = lax.iota(jnp.int32, _L)
    zeros_i = jnp.zeros((_L,), jnp.int32)
    ones_i = jnp.ones((_L,), jnp.int32)
    zeros_f = jnp.zeros((_L,), jnp.float32)
    safe = (safe_a, safe_b)
    rows = (rows_a, rows_b)
    tbuf = (tbuf_a, tbuf_b)
    gsem = (gsem_a, gsem_b)
    osem = (osem_a, osem_b)

    pltpu.async_copy(
        idx_hbm.at[pl.ds(wid * per_w_idx, per_w_idx)], idx_all, isem).wait()

    def sanitize(g, slot):
      base = g * chunk
      cnt = zeros_i
      for gr in range(groups):
        v = idx_all[pl.ds(base + gr * _L, _L)]
        oov = v >= vocab
        safe[slot][gr // 8, pl.ds((gr % 8) * _L, _L)] = jnp.where(
            oov, zeros_i, v)
        cnt = cnt + jnp.where(oov, ones_i, zeros_i)
      return jnp.sum(cnt)

    def fire_gather(slot):
      for j in range(4):
        pltpu.async_copy(
            table_hbm.at[safe[slot].at[j]],
            rows[slot].at[pl.ds(j * 128, 128)], gsem[slot])

    def wait_gather(slot):
      for j in range(4):
        pltpu.make_async_copy(
            table_hbm.at[safe[slot].at[j]],
            rows[slot].at[pl.ds(j * 128, 128)], gsem[slot]).wait()

    def zero_oov(g, slot, tot):
      @pl.when(tot > 0)
      def _():
        base = g * chunk
        def zgrp(gr, zc):
          v = idx_all[pl.ds(base + gr * _L, _L)]
          inval = v >= vocab
          rr = gr * _L + iota
          for col in range(dim):
            plsc.store_scatter(
                rows[slot], [rr, jnp.full((_L,), col, jnp.int32)],
                zeros_f, mask=inval)
          return zc
        lax.fori_loop(0, groups, zgrp, 0)

    def transpose(slot):
      for ft in range(ftiles):
        def tbody(i, carry, ft=ft):
          bts = i >> 6
          fr = (i >> 3) & 7
          bcg = i & 7
          rr = bts * 128 + bcg * _L + iota
          colv = zeros_i + (ft * 8 + fr)
          vec = plsc.load_gather(rows[slot], [rr, colv])
          tbuf[slot][ft, bts, pl.ds(fr * 128 + bcg * _L, _L)] = vec
          return carry
        lax.fori_loop(0, 256, tbody, 0)

    def fire_out(g, slot):
      su = wid * su_per_w + g
      h = su // nb4
      bt0 = (su % nb4) * 4
      for ft in range(ftiles):
        pltpu.async_copy(
            tbuf[slot].at[ft], out_hbm.at[h, ft, pl.ds(bt0, 4)], osem[slot])

    def wait_out(slot):
      for ft in range(ftiles):
        pltpu.make_async_copy(
            tbuf[slot].at[ft], out_hbm.at[0, ft, pl.ds(0, 4)],
            osem[slot]).wait()

    def prefetch(g, slot):
      tot = sanitize(g, slot)
      fire_gather(slot)
      return tot

    prefetch(0, 0)

    def consume(g, i, slot):
      wait_gather(slot)
      base = g * chunk
      cnt = zeros_i
      for gr in range(groups):
        v = idx_all[pl.ds(base + gr * _L, _L)]
        cnt = cnt + jnp.where(v >= vocab, ones_i, zeros_i)
      zero_oov(g, slot, jnp.sum(cnt))

      @pl.when(i >= 1)
      def _():
        wait_out(slot)
      transpose(slot)
      fire_out(g, slot)

    def pair_body(i, carry):
      g0 = 2 * i
      g1 = g0 + 1
      prefetch(g1, 1)
      consume(g0, i, 0)

      @pl.when(i < su_per_w // 2 - 1)
      def _():
        prefetch(g0 + 2, 0)
      consume(g1, i, 1)
      return carry

    lax.fori_loop(0, su_per_w // 2, pair_body, 0)
    wait_out(0)
    wait_out(1)

  return gather_kernel


def kernel(inputs, embeddings):
  b, h = inputs.shape
  vocab, dim = embeddings.shape
  fn = _build(vocab, b, h, dim)
  idx_flat = inputs.T.reshape(b * h)
  out = fn(idx_flat, embeddings)
  out5 = out.reshape(h, dim // 8, b // 128, 8, 128)
  return out5.transpose(2, 4, 0, 1, 3).reshape(b, h, dim)


# speedup vs baseline: 1.7068x; 1.0022x over previous
import functools

import jax
import jax.numpy as jnp
from jax import lax
from jax.experimental import pallas as pl
from jax.experimental.pallas import tpu as pltpu
from jax.experimental.pallas import tpu_sc as plsc

_L = 16


def _build(vocab, batch, hist, dim):
  n_workers = 32
  chunk = 512
  groups = chunk // _L
  nbt = batch // 128
  nb4 = nbt // 4
  n_su = hist * nb4
  su_per_w = n_su // n_workers
  per_w_idx = su_per_w * chunk
  ftiles = dim // 8

  mesh = plsc.VectorSubcoreMesh(core_axis_name="c", subcore_axis_name="s")

  @functools.partial(
      pl.kernel,
      mesh=mesh,
      out_type=jax.ShapeDtypeStruct((hist, ftiles, nbt, 8 * 128), jnp.float32),
      compiler_params=pltpu.CompilerParams(
          needs_layout_passes=False, use_tc_tiling_on_sc=False),
      scratch_types=[
          pltpu.VMEM((per_w_idx,), jnp.int32),
          pltpu.VMEM((4, 128), jnp.int32),
          pltpu.VMEM((4, 128), jnp.int32),
          pltpu.VMEM((chunk, dim), jnp.float32),
          pltpu.VMEM((chunk, dim), jnp.float32),
          pltpu.VMEM((ftiles, 4, 1024), jnp.float32),
          pltpu.VMEM((ftiles, 4, 1024), jnp.float32),
          pltpu.SemaphoreType.DMA,
          pltpu.SemaphoreType.DMA,
          pltpu.SemaphoreType.DMA,
          pltpu.SemaphoreType.DMA,
          pltpu.SemaphoreType.DMA,
      ],
  )
  def gather_kernel(idx_hbm, table_hbm, out_hbm, idx_all, safe_a, safe_b,
                    rows_a, rows_b, tbuf_a, tbuf_b, isem, gsem_a, gsem_b,
                    osem_a, osem_b):
    nc = 2
    wid = lax.axis_index("s") * nc + lax.axis_index("c")
    iota = lax.iota(jnp.int32, _L)
    zeros_i = jnp.zeros((_L,), jnp.int32)
    ones_i = jnp.ones((_L,), jnp.int32)
    zeros_f = jnp.zeros((_L,), jnp.float32)
    safe = (safe_a, safe_b)
    rows = (rows_a, rows_b)
    tbuf = (tbuf_a, tbuf_b)
    gsem = (gsem_a, gsem_b)
    osem = (osem_a, osem_b)

    pltpu.async_copy(
        idx_hbm.at[pl.ds(wid * per_w_idx, per_w_idx)], idx_all, isem).wait()

    def sanitize(g, slot):
      base = g * chunk
      for gr in range(groups):
        v = idx_all[pl.ds(base + gr * _L, _L)]
        safe[slot][gr // 8, pl.ds((gr % 8) * _L, _L)] = jnp.where(
            v >= vocab, zeros_i, v)

    def fire_gather(slot):
      for j in range(4):
        pltpu.async_copy(
            table_hbm.at[safe[slot].at[j]],
            rows[slot].at[pl.ds(j * 128, 128)], gsem[slot])

    def wait_gather(slot):
      for j in range(4):
        pltpu.make_async_copy(
            table_hbm.at[safe[slot].at[j]],
            rows[slot].at[pl.ds(j * 128, 128)], gsem[slot]).wait()

    def zero_oov(g, slot, tot):
      @pl.when(tot > 0)
      def _():
        base = g * chunk
        def zgrp(gr, zc):
          v = idx_all[pl.ds(base + gr * _L, _L)]
          inval = v >= vocab
          rr = gr * _L + iota
          for col in range(dim):
            plsc.store_scatter(
                rows[slot], [rr, jnp.full((_L,), col, jnp.int32)],
                zeros_f, mask=inval)
          return zc
        lax.fori_loop(0, groups, zgrp, 0)

    def transpose(slot):
      for ft in range(ftiles):
        def tbody(i, carry, ft=ft):
          bts = i >> 3
          fr = i & 7
          rr0 = (bts << 7) + iota
          colv = zeros_i + ((ft << 3) + fr)
          off0 = fr << 7
          for bcg in range(8):
            vec = plsc.load_gather(rows[slot], [rr0 + bcg * _L, colv])
            tbuf[slot][ft, bts, pl.ds(off0 + bcg * _L, _L)] = vec
          return carry
        lax.fori_loop(0, 32, tbody, 0)

    def fire_out(g, slot):
      su = wid * su_per_w + g
      h = su // nb4
      bt0 = (su % nb4) * 4
      for ft in range(ftiles):
        pltpu.async_copy(
            tbuf[slot].at[ft], out_hbm.at[h, ft, pl.ds(bt0, 4)], osem[slot])

    def wait_out(slot):
      for ft in range(ftiles):
        pltpu.make_async_copy(
            tbuf[slot].at[ft], out_hbm.at[0, ft, pl.ds(0, 4)],
            osem[slot]).wait()

    def prefetch(g, slot):
      sanitize(g, slot)
      fire_gather(slot)

    prefetch(0, 0)

    def consume(g, i, slot):
      wait_gather(slot)
      base = g * chunk
      cnt = zeros_i
      for gr in range(groups):
        v = idx_all[pl.ds(base + gr * _L, _L)]
        cnt = cnt + jnp.where(v >= vocab, ones_i, zeros_i)
      zero_oov(g, slot, jnp.sum(cnt))

      @pl.when(i >= 1)
      def _():
        wait_out(slot)
      transpose(slot)
      fire_out(g, slot)

    def pair_body(i, carry):
      g0 = 2 * i
      g1 = g0 + 1
      prefetch(g1, 1)
      consume(g0, i, 0)

      @pl.when(i < su_per_w // 2 - 1)
      def _():
        prefetch(g0 + 2, 0)
      consume(g1, i, 1)
      return carry

    lax.fori_loop(0, su_per_w // 2, pair_body, 0)
    wait_out(0)
    wait_out(1)

  return gather_kernel


def kernel(inputs, embeddings):
  b, h = inputs.shape
  vocab, dim = embeddings.shape
  fn = _build(vocab, b, h, dim)
  idx_flat = inputs.T.reshape(b * h)
  out = fn(idx_flat, embeddings)
  out5 = out.reshape(h, dim // 8, b // 128, 8, 128)
  return out5.transpose(2, 4, 0, 1, 3).reshape(b, h, dim)


# speedup vs baseline: 1.8409x; 1.0786x over previous
import functools

import jax
import jax.numpy as jnp
from jax import lax
from jax.experimental import pallas as pl
from jax.experimental.pallas import tpu as pltpu
from jax.experimental.pallas import tpu_sc as plsc

_L = 16


def _build(vocab, batch, hist, dim):
  n_workers = 32
  chunk = 512
  groups = chunk // _L
  nbt = batch // 128
  nb4 = nbt // 4
  n_su = hist * nb4
  su_per_w = n_su // n_workers
  per_w_idx = su_per_w * chunk
  ftiles = dim // 8

  mesh = plsc.VectorSubcoreMesh(core_axis_name="c", subcore_axis_name="s")

  @functools.partial(
      pl.kernel,
      mesh=mesh,
      out_type=jax.ShapeDtypeStruct((hist, ftiles * nbt * 1024), jnp.float32),
      compiler_params=pltpu.CompilerParams(
          needs_layout_passes=False, use_tc_tiling_on_sc=False),
      scratch_types=[
          pltpu.VMEM((per_w_idx,), jnp.int32),
          pltpu.VMEM((4, 128), jnp.int32),
          pltpu.VMEM((4, 128), jnp.int32),
          pltpu.VMEM((chunk, dim), jnp.float32),
          pltpu.VMEM((chunk, dim), jnp.float32),
          pltpu.VMEM((ftiles * 4096,), jnp.float32),
          pltpu.VMEM((ftiles * 4096,), jnp.float32),
          pltpu.VMEM((chunk,), jnp.int32),
          pltpu.VMEM((chunk,), jnp.int32),
          pltpu.SemaphoreType.DMA,
          pltpu.SemaphoreType.DMA,
          pltpu.SemaphoreType.DMA,
          pltpu.SemaphoreType.DMA,
          pltpu.SemaphoreType.DMA,
      ],
  )
  def gather_kernel(idx_hbm, table_hbm, out_hbm, idx_all, safe_a, safe_b,
                    rows_a, rows_b, tbuf_a, tbuf_b, ccbuf, svbuf, isem,
                    gsem_a, gsem_b, osem_a, osem_b):
    nc = 2
    wid = lax.axis_index("s") * nc + lax.axis_index("c")
    iota = lax.iota(jnp.int32, _L)
    zeros_i = jnp.zeros((_L,), jnp.int32)
    ones_i = jnp.ones((_L,), jnp.int32)
    zeros_f = jnp.zeros((_L,), jnp.float32)
    safe = (safe_a, safe_b)
    rows = (rows_a, rows_b)
    tbuf = (tbuf_a, tbuf_b)
    gsem = (gsem_a, gsem_b)
    osem = (osem_a, osem_b)

    pltpu.async_copy(
        idx_hbm.at[pl.ds(wid * per_w_idx, per_w_idx)], idx_all, isem).wait()

    for f0 in range(32):
      f = (f0 + iota) & 31
      ccbuf[pl.ds(f0 * _L, _L)] = f
      svbuf[pl.ds(f0 * _L, _L)] = (
          (f >> 3) * 4096 + (f & 7) * 128 + iota)

    def sanitize(g, slot):
      base = g * chunk
      for gr in range(groups):
        v = idx_all[pl.ds(base + gr * _L, _L)]
        safe[slot][gr // 8, pl.ds((gr % 8) * _L, _L)] = jnp.where(
            v >= vocab, zeros_i, v)

    def fire_gather(slot):
      for j in range(4):
        pltpu.async_copy(
            table_hbm.at[safe[slot].at[j]],
            rows[slot].at[pl.ds(j * 128, 128)], gsem[slot])

    def wait_gather(slot):
      for j in range(4):
        pltpu.make_async_copy(
            table_hbm.at[safe[slot].at[j]],
            rows[slot].at[pl.ds(j * 128, 128)], gsem[slot]).wait()

    def zero_oov(g, slot, tot):
      @pl.when(tot > 0)
      def _():
        base = g * chunk
        def zgrp(gr, zc):
          v = idx_all[pl.ds(base + gr * _L, _L)]
          inval = v >= vocab
          rr = gr * _L + iota
          for col in range(dim):
            plsc.store_scatter(
                rows[slot], [rr, jnp.full((_L,), col, jnp.int32)],
                zeros_f, mask=inval)
          return zc
        lax.fori_loop(0, groups, zgrp, 0)

    def transpose(slot):
      def tr_body(rg, carry):
        rr = (rg * _L) + iota
        dyn_s = (rg >> 3) * 1024 + (rg & 7) * _L
        for f0 in range(32):
          ccv = ccbuf[pl.ds(f0 * _L, _L)]
          svv = svbuf[pl.ds(f0 * _L, _L)]
          vec = plsc.load_gather(rows[slot], [rr, ccv])
          plsc.store_scatter(tbuf[slot], [svv + dyn_s], vec)
        return carry
      lax.fori_loop(0, 32, tr_body, 0)

    def fire_out(g, slot):
      su = wid * su_per_w + g
      h = su // nb4
      o0 = (su % nb4) * 4096
      for ft in range(ftiles):
        pltpu.async_copy(
            tbuf[slot].at[pl.ds(ft * 4096, 4096)],
            out_hbm.at[h, pl.ds(ft * nbt * 1024 + o0, 4096)], osem[slot])

    def wait_out(slot):
      for ft in range(ftiles):
        pltpu.make_async_copy(
            tbuf[slot].at[pl.ds(ft * 4096, 4096)],
            out_hbm.at[0, pl.ds(ft * nbt * 1024, 4096)], osem[slot]).wait()

    def prefetch(g, slot):
      sanitize(g, slot)
      fire_gather(slot)

    prefetch(0, 0)

    def consume(g, i, slot):
      wait_gather(slot)
      base = g * chunk
      cnt = zeros_i
      for gr in range(groups):
        v = idx_all[pl.ds(base + gr * _L, _L)]
        cnt = cnt + jnp.where(v >= vocab, ones_i, zeros_i)
      zero_oov(g, slot, jnp.sum(cnt))

      @pl.when(i >= 1)
      def _():
        wait_out(slot)
      transpose(slot)
      fire_out(g, slot)

    def pair_body(i, carry):
      g0 = 2 * i
      g1 = g0 + 1
      prefetch(g1, 1)
      consume(g0, i, 0)

      @pl.when(i < su_per_w // 2 - 1)
      def _():
        prefetch(g0 + 2, 0)
      consume(g1, i, 1)
      return carry

    lax.fori_loop(0, su_per_w // 2, pair_body, 0)
    wait_out(0)
    wait_out(1)

  return gather_kernel


def kernel(inputs, embeddings):
  b, h = inputs.shape
  vocab, dim = embeddings.shape
  fn = _build(vocab, b, h, dim)
  idx_flat = inputs.T.reshape(b * h)
  out = fn(idx_flat, embeddings)
  out5 = out.reshape(h, dim // 8, b // 128, 8, 128)
  return out5.transpose(2, 4, 0, 1, 3).reshape(b, h, dim)
